# baseline (speedup 1.0000x reference)
import functools

import jax
import jax.numpy as jnp
from jax.experimental import pallas as pl
from jax.experimental.pallas import tpu as pltpu


def _relu(v):
    return jnp.maximum(v, 0.0)


def _dot(a, b, precision=jax.lax.Precision.DEFAULT):
    return jnp.dot(a, b, preferred_element_type=jnp.float32,
                   precision=precision)


def _ds_pass_body(nsteps, nseg, with_out, refs):
    if with_out:
        (x_ref, bo_ref, bs_ref, u_other_ref,
         w1, b1, w2, b2, w3, b3,
         r1, rb1, r2, rb2, r3, rb3,
         m1, mb1, m2, mb2, m3, mb3,
         u_new_ref, out_ref, acc_ref) = refs
    else:
        (x_ref, bo_ref, bs_ref, u_other_ref,
         w1, b1, w2, b2, w3, b3,
         r1, rb1, r2, rb2, r3, rb3,
         u_new_ref, acc_ref) = refs

    step = pl.program_id(0)
    blk = x_ref.shape[0]

    u_other = u_other_ref[...]

    def _half(off, sub):
        bo = bo_ref[off:off + sub, :]
        oh_g = (bo == jax.lax.broadcasted_iota(jnp.int32, (sub, nseg), 1)
                ).astype(jnp.float32)
        ug = _dot(oh_g, u_other)

        xc = jnp.concatenate([x_ref[off:off + sub, :].astype(jnp.bfloat16),
                              ug.astype(jnp.bfloat16)], axis=1)
        h = _relu(_dot(xc, w1[...]) + b1[...])
        h = _relu(_dot(h, w2[...]) + b2[...])
        h = _dot(h, w3[...]) + b3[...]

        bs = bs_ref[0, :, off:off + sub]
        oh_s = (bs == jax.lax.broadcasted_iota(jnp.int32, (nseg, sub), 0)
                ).astype(jnp.float32)
        h_hi = h.astype(jnp.bfloat16).astype(jnp.float32)
        h_rem = h - h_hi
        h_mid = h_rem.astype(jnp.bfloat16).astype(jnp.float32)
        return (_dot(oh_s, h_hi) + _dot(oh_s, h_mid)
                + _dot(oh_s, h_rem - h_mid))

    sub = blk // 2
    part = _half(0, sub) + _half(sub, sub)

    @pl.when(step == 0)
    def _():
        acc_ref[...] = part

    @pl.when(step > 0)
    def _():
        acc_ref[...] += part

    @pl.when(step == nsteps - 1)
    def _():
        agg = acc_ref[...]
        r = _relu(_dot(agg, r1[...]) + rb1[...])
        r = _relu(_dot(r, r2[...]) + rb2[...])
        u_new = _dot(r, r3[...]) + rb3[...]
        u_new_ref[...] = u_new
        if with_out:
            mc = jnp.concatenate([u_other, u_new], axis=1)
            m = _relu(_dot(mc, m1[...]) + mb1[...])
            m = _relu(_dot(m, m2[...]) + mb2[...])
            out_ref[...] = _dot(m, m3[...]) + mb3[...]


def _flatten_mlp(mlp_params):
    out = []
    for (w, b) in mlp_params:
        out.extend([w, b.reshape(1, -1)])
    return out


def _pick_block(n):
    for blk in (2000, 1000, 500, 1000 // 8 * 8):
        if n % blk == 0:
            return blk, 0
    blk = 2000
    pad = (-n) % blk
    return blk, pad


def _ds_pass(x, batch_self, batch_other, u_other, phi, rho, mlp=None):
    n, fx = x.shape
    nseg, fu = u_other.shape
    h = phi[-1][0].shape[1]
    blk, pad = _pick_block(n)
    if pad:
        x = jnp.pad(x, ((0, pad), (0, 0)))
        batch_self = jnp.pad(batch_self, (0, pad), constant_values=nseg)
        batch_other = jnp.pad(batch_other, (0, pad), constant_values=nseg)
        n = n + pad
    nsteps = n // blk

    bo_col = batch_other.reshape(n, 1)
    bs_row = batch_self.reshape(nsteps, 1, blk)

    phi_refs = _flatten_mlp(phi)
    phi_refs[0] = phi_refs[0].astype(jnp.bfloat16)
    rho_refs = _flatten_mlp(rho)
    mlp_refs = _flatten_mlp(mlp) if mlp is not None else []
    with_out = mlp is not None

    const = lambda *shape: pl.BlockSpec(shape, lambda i: (0,) * len(shape))
    in_specs = (
        [pl.BlockSpec((blk, fx), lambda i: (i, 0)),
         pl.BlockSpec((blk, 1), lambda i: (i, 0)),
         pl.BlockSpec((1, 1, blk), lambda i: (i, 0, 0)),
         const(nseg, fu)]
        + [const(*w.shape) for w in phi_refs]
        + [const(*w.shape) for w in rho_refs]
        + [const(*w.shape) for w in mlp_refs]
    )

    fout = mlp[-1][0].shape[1] if with_out else 0
    out_shape = [jax.ShapeDtypeStruct((nseg, fu), jnp.float32)]
    out_specs = [const(nseg, fu)]
    if with_out:
        out_shape.append(jax.ShapeDtypeStruct((nseg, fout), jnp.float32))
        out_specs.append(const(nseg, fout))

    body = lambda *refs: _ds_pass_body(nsteps, nseg, with_out, refs)
    res = pl.pallas_call(
        body,
        grid=(nsteps,),
        in_specs=in_specs,
        out_specs=out_specs,
        out_shape=out_shape,
        scratch_shapes=[pltpu.VMEM((nseg, h), jnp.float32)],
    )(x, bo_col, bs_row, u_other, *phi_refs, *rho_refs, *mlp_refs)
    if with_out:
        return res[0], res[1]
    return res[0]


def kernel(x1, batch1, u1, x2, batch2, u2, params):
    batch1 = batch1.astype(jnp.int32)
    batch2 = batch2.astype(jnp.int32)
    x1 = x1.astype(jnp.float32)
    x2 = x2.astype(jnp.float32)
    u2 = u2.astype(jnp.float32)
    phi1, rho1 = params["ds1"]
    phi2, rho2 = params["ds2"]
    mlp = params["mlp"]
    niter = 2
    outs = []
    for _ in range(niter):
        u1 = _ds_pass(x1, batch1, batch2, u2, phi1, rho1)
        u2, out = _ds_pass(x2, batch2, batch1, u1, phi2, rho2, mlp=mlp)
        outs.append(out)
    return tuple(outs)


# speedup vs baseline: 1.3218x; 1.3218x over previous
import functools

import jax
import jax.numpy as jnp
from jax.experimental import pallas as pl
from jax.experimental.pallas import tpu as pltpu


def _relu(v):
    return jnp.maximum(v, 0.0)


def _dot(a, b, precision=jax.lax.Precision.DEFAULT):
    return jnp.dot(a, b, preferred_element_type=jnp.float32,
                   precision=precision)


def _ds_pass_body(nsteps, nseg, with_out, refs):
    if with_out:
        (x_ref, bo_ref, bs_ref, u_other_ref,
         w1, b1, w2, b2, w3, b3,
         r1, rb1, r2, rb2, r3, rb3,
         m1, mb1, m2, mb2, m3, mb3,
         u_new_ref, out_ref, acc_ref) = refs
    else:
        (x_ref, bo_ref, bs_ref, u_other_ref,
         w1, b1, w2, b2, w3, b3,
         r1, rb1, r2, rb2, r3, rb3,
         u_new_ref, acc_ref) = refs

    step = pl.program_id(0)
    blk = x_ref.shape[0]

    u_other = u_other_ref[...]

    bo = bo_ref[...]
    oh_g = (bo == jax.lax.broadcasted_iota(jnp.int32, (blk, nseg), 1)
            ).astype(jnp.float32)
    ug = _dot(oh_g, u_other)

    xc = jnp.concatenate([x_ref[...], ug], axis=1)
    h = _relu(_dot(xc, w1[...]) + b1[...])
    h = _relu(_dot(h, w2[...]) + b2[...])
    h = _dot(h, w3[...]) + b3[...]

    bs = bs_ref[0]
    oh_s = (bs == jax.lax.broadcasted_iota(jnp.int32, (nseg, blk), 0)
            ).astype(jnp.float32)
    h_hi = h.astype(jnp.bfloat16).astype(jnp.float32)
    h_rem = h - h_hi
    h_mid = h_rem.astype(jnp.bfloat16).astype(jnp.float32)
    part = (_dot(oh_s, h_hi) + _dot(oh_s, h_mid)
            + _dot(oh_s, h_rem - h_mid))

    @pl.when(step == 0)
    def _():
        acc_ref[...] = part

    @pl.when(step > 0)
    def _():
        acc_ref[...] += part

    @pl.when(step == nsteps - 1)
    def _():
        agg = acc_ref[...]
        r = _relu(_dot(agg, r1[...]) + rb1[...])
        r = _relu(_dot(r, r2[...]) + rb2[...])
        u_new = _dot(r, r3[...]) + rb3[...]
        u_new_ref[...] = u_new
        if with_out:
            mc = jnp.concatenate([u_other, u_new], axis=1)
            m = _relu(_dot(mc, m1[...]) + mb1[...])
            m = _relu(_dot(m, m2[...]) + mb2[...])
            out_ref[...] = _dot(m, m3[...]) + mb3[...]


def _flatten_mlp(mlp_params):
    out = []
    for (w, b) in mlp_params:
        out.extend([w, b.reshape(1, -1)])
    return out


def _pick_block(n):
    for blk in (4000, 2000, 1000, 500, 200):
        if n % blk == 0:
            return blk, 0
    blk = 4000
    pad = (-n) % blk
    return blk, pad


def _ds_pass(x, batch_self, batch_other, u_other, phi, rho, mlp=None):
    n, fx = x.shape
    nseg, fu = u_other.shape
    h = phi[-1][0].shape[1]
    blk, pad = _pick_block(n)
    if pad:
        x = jnp.pad(x, ((0, pad), (0, 0)))
        batch_self = jnp.pad(batch_self, (0, pad), constant_values=nseg)
        batch_other = jnp.pad(batch_other, (0, pad), constant_values=nseg)
        n = n + pad
    nsteps = n // blk

    bo_col = batch_other.reshape(n, 1)
    bs_row = batch_self.reshape(nsteps, 1, blk)

    phi_refs = _flatten_mlp(phi)
    rho_refs = _flatten_mlp(rho)
    mlp_refs = _flatten_mlp(mlp) if mlp is not None else []
    with_out = mlp is not None

    const = lambda *shape: pl.BlockSpec(shape, lambda i: (0,) * len(shape))
    in_specs = (
        [pl.BlockSpec((blk, fx), lambda i: (i, 0)),
         pl.BlockSpec((blk, 1), lambda i: (i, 0)),
         pl.BlockSpec((1, 1, blk), lambda i: (i, 0, 0)),
         const(nseg, fu)]
        + [const(*w.shape) for w in phi_refs]
        + [const(*w.shape) for w in rho_refs]
        + [const(*w.shape) for w in mlp_refs]
    )

    fout = mlp[-1][0].shape[1] if with_out else 0
    out_shape = [jax.ShapeDtypeStruct((nseg, fu), jnp.float32)]
    out_specs = [const(nseg, fu)]
    if with_out:
        out_shape.append(jax.ShapeDtypeStruct((nseg, fout), jnp.float32))
        out_specs.append(const(nseg, fout))

    body = lambda *refs: _ds_pass_body(nsteps, nseg, with_out, refs)
    res = pl.pallas_call(
        body,
        grid=(nsteps,),
        in_specs=in_specs,
        out_specs=out_specs,
        out_shape=out_shape,
        scratch_shapes=[pltpu.VMEM((nseg, h), jnp.float32)],
    )(x, bo_col, bs_row, u_other, *phi_refs, *rho_refs, *mlp_refs)
    if with_out:
        return res[0], res[1]
    return res[0]


def kernel(x1, batch1, u1, x2, batch2, u2, params):
    batch1 = batch1.astype(jnp.int32)
    batch2 = batch2.astype(jnp.int32)
    x1 = x1.astype(jnp.float32)
    x2 = x2.astype(jnp.float32)
    u2 = u2.astype(jnp.float32)
    phi1, rho1 = params["ds1"]
    phi2, rho2 = params["ds2"]
    mlp = params["mlp"]
    niter = 2
    outs = []
    for _ in range(niter):
        u1 = _ds_pass(x1, batch1, batch2, u2, phi1, rho1)
        u2, out = _ds_pass(x2, batch2, batch1, u1, phi2, rho2, mlp=mlp)
        outs.append(out)
    return tuple(outs)
